# baseline (speedup 1.0000x reference)
import jax
import jax.numpy as jnp
from jax.experimental import pallas as pl
from jax.experimental.pallas import tpu as pltpu

N = 1000000
BROWS, BCOLS = 8, 4096
CHUNK = BROWS * BCOLS
FULL_STEPS = N // CHUNK
LAST_OFF = FULL_STEPS * CHUNK
M_MAIN = 999936
LAST_ROWS = 4
LAST_PART = M_MAIN - (LAST_OFF + LAST_ROWS * BCOLS)
TAIL = N - M_MAIN
GRID = FULL_STEPS + 1
SCOL = 512

_KS0 = 0
_KS1 = 42
_KS2 = _KS0 ^ _KS1 ^ 0x1BD11BDA
_ROT = ((13, 15, 26, 6), (17, 29, 16, 24))


def _rotl(x, d):
    return (x << jnp.uint32(d)) | (x >> jnp.uint32(32 - d))


def _threefry_bits(c2):
    ks = (jnp.uint32(_KS0), jnp.uint32(_KS1), jnp.uint32(_KS2))
    x0 = jnp.full(c2.shape, _KS0, jnp.uint32)
    x1 = c2 + ks[1]

    def rounds(x0, x1, rs):
        for r in rs:
            x0 = x0 + x1
            x1 = _rotl(x1, r)
            x1 = x0 ^ x1
        return x0, x1

    x0, x1 = rounds(x0, x1, _ROT[0])
    x0, x1 = x0 + ks[1], x1 + ks[2] + jnp.uint32(1)
    x0, x1 = rounds(x0, x1, _ROT[1])
    x0, x1 = x0 + ks[2], x1 + ks[0] + jnp.uint32(2)
    x0, x1 = rounds(x0, x1, _ROT[0])
    x0, x1 = x0 + ks[0], x1 + ks[1] + jnp.uint32(3)
    x0, x1 = rounds(x0, x1, _ROT[1])
    x0, x1 = x0 + ks[1], x1 + ks[2] + jnp.uint32(4)
    x0, x1 = rounds(x0, x1, _ROT[0])
    x0, x1 = x0 + ks[2], x1 + ks[0] + jnp.uint32(5)
    return x0 ^ x1


def _gumbel_z(pos, w):
    bits = _threefry_bits(pos.astype(jnp.uint32))
    fbits = (bits >> jnp.uint32(9)) | jnp.uint32(0x3F800000)
    f = jax.lax.bitcast_convert_type(fbits, jnp.float32) - jnp.float32(1.0)
    u = jnp.maximum(f, jnp.float32(1.1754943508222875e-38))
    return -jnp.log(-jnp.log(u)) + jnp.log(w)


def _chunk_off(j):
    return jnp.where(j < FULL_STEPS, j * CHUNK, LAST_OFF)


def _body(w_hbm, wt_ref, out_ref, buf, m_acc, idx_acc, sems):
    j = pl.program_id(0)
    slot = jax.lax.rem(j, 2)

    def row_copy(off, slot_ix, r, length=BCOLS):
        src = w_hbm.at[pl.ds(off + r * BCOLS, length)]
        return pltpu.make_async_copy(src, buf.at[slot_ix, r, pl.ds(0, length)],
                                     sems.at[slot_ix])

    def start_full(off, slot_ix):
        for r in range(BROWS):
            row_copy(off, slot_ix, r).start()

    def start_last(slot_ix):
        for r in range(LAST_ROWS):
            row_copy(LAST_OFF, slot_ix, r).start()
        row_copy(LAST_OFF, slot_ix, LAST_ROWS, LAST_PART).start()

    @pl.when(j == 0)
    def _():
        start_full(0, 0)

    @pl.when(j + 1 < FULL_STEPS)
    def _():
        start_full((j + 1) * CHUNK, 1 - slot)

    @pl.when(j + 1 == FULL_STEPS)
    def _():
        start_last(1 - slot)

    @pl.when(j < FULL_STEPS)
    def _():
        for r in range(BROWS):
            row_copy(0, slot, r).wait()

    @pl.when(j == FULL_STEPS)
    def _():
        for r in range(LAST_ROWS):
            row_copy(0, slot, r).wait()
        row_copy(0, slot, LAST_ROWS, LAST_PART).wait()

    row = jax.lax.broadcasted_iota(jnp.int32, (BROWS, SCOL), 0)
    col = jax.lax.broadcasted_iota(jnp.int32, (BROWS, SCOL), 1)
    pos0 = _chunk_off(j) + row * BCOLS + col

    def step(i, carry):
        m_vec, idx_vec = carry
        pos = pos0 + i * SCOL
        w = buf[slot, :, pl.ds(i * SCOL, SCOL)]
        z = _gumbel_z(pos, w)
        z = jnp.where(pos < M_MAIN, z, -jnp.inf)
        upd = z > m_vec
        m_vec = jnp.where(upd, z, m_vec)
        idx_vec = jnp.where(upd, pos, idx_vec)
        return m_vec, idx_vec

    m0 = jnp.where(j == 0, jnp.full((BROWS, SCOL), -jnp.inf, jnp.float32),
                   m_acc[...])
    i0 = jnp.where(j == 0, jnp.full((BROWS, SCOL), 2**31 - 1, jnp.int32),
                   idx_acc[...])
    m_vec, idx_vec = jax.lax.fori_loop(
        0, BCOLS // SCOL, step, (m0, i0), unroll=4)
    m_acc[...] = m_vec
    idx_acc[...] = idx_vec

    @pl.when(j == GRID - 1)
    def _():
        m1 = jnp.max(m_vec)
        bi1 = jnp.min(jnp.where(m_vec == m1, idx_vec, jnp.int32(2**31 - 1)))
        trow = jax.lax.broadcasted_iota(jnp.int32, (8, 128), 0)
        tcol = jax.lax.broadcasted_iota(jnp.int32, (8, 128), 1)
        tpos = M_MAIN + trow * 128 + tcol
        zt = _gumbel_z(tpos, wt_ref[...])
        m2 = jnp.max(zt)
        bi2 = jnp.min(jnp.where(zt == m2, tpos, jnp.int32(2**31 - 1)))
        out_ref[0] = jnp.where(m2 > m1, bi2, bi1)


def kernel(weights):
    wt = jnp.pad(weights[M_MAIN:], (0, 8 * 128 - TAIL)).reshape(8, 128)
    idx = pl.pallas_call(
        _body,
        grid=(GRID,),
        in_specs=[pl.BlockSpec(memory_space=pltpu.HBM),
                  pl.BlockSpec((8, 128), lambda j: (0, 0))],
        out_specs=pl.BlockSpec(memory_space=pltpu.SMEM),
        out_shape=jax.ShapeDtypeStruct((1,), jnp.int32),
        scratch_shapes=[
            pltpu.VMEM((2, BROWS, BCOLS), jnp.float32),
            pltpu.VMEM((BROWS, SCOL), jnp.float32),
            pltpu.VMEM((BROWS, SCOL), jnp.int32),
            pltpu.SemaphoreType.DMA((2,)),
        ],
    )(weights, wt)
    return idx


# speedup vs baseline: 1.0241x; 1.0241x over previous
import jax
import jax.numpy as jnp
from jax.experimental import pallas as pl
from jax.experimental.pallas import tpu as pltpu

N = 1000000
BROWS, BCOLS = 8, 4096
CHUNK = BROWS * BCOLS
FULL_STEPS = N // CHUNK
LAST_OFF = FULL_STEPS * CHUNK
M_MAIN = 999936
LAST_ROWS = 4
LAST_PART = M_MAIN - (LAST_OFF + LAST_ROWS * BCOLS)
TAIL = N - M_MAIN
GRID = FULL_STEPS + 1
SCOL = 512

_KS0 = 0
_KS1 = 42
_KS2 = _KS0 ^ _KS1 ^ 0x1BD11BDA
_ROT = ((13, 15, 26, 6), (17, 29, 16, 24))


def _rotl(x, d):
    return (x << jnp.uint32(d)) | (x >> jnp.uint32(32 - d))


def _threefry_bits(c2):
    ks = (jnp.uint32(_KS0), jnp.uint32(_KS1), jnp.uint32(_KS2))
    x0 = jnp.full(c2.shape, _KS0, jnp.uint32)
    x1 = c2 + ks[1]

    def rounds(x0, x1, rs):
        for r in rs:
            x0 = x0 + x1
            x1 = _rotl(x1, r)
            x1 = x0 ^ x1
        return x0, x1

    x0, x1 = rounds(x0, x1, _ROT[0])
    x0, x1 = x0 + ks[1], x1 + ks[2] + jnp.uint32(1)
    x0, x1 = rounds(x0, x1, _ROT[1])
    x0, x1 = x0 + ks[2], x1 + ks[0] + jnp.uint32(2)
    x0, x1 = rounds(x0, x1, _ROT[0])
    x0, x1 = x0 + ks[0], x1 + ks[1] + jnp.uint32(3)
    x0, x1 = rounds(x0, x1, _ROT[1])
    x0, x1 = x0 + ks[1], x1 + ks[2] + jnp.uint32(4)
    x0, x1 = rounds(x0, x1, _ROT[0])
    x0, x1 = x0 + ks[2], x1 + ks[0] + jnp.uint32(5)
    return x0 ^ x1


def _gumbel_z(pos, w):
    bits = _threefry_bits(pos.astype(jnp.uint32))
    fbits = (bits >> jnp.uint32(9)) | jnp.uint32(0x3F800000)
    f = jax.lax.bitcast_convert_type(fbits, jnp.float32) - jnp.float32(1.0)
    u = jnp.maximum(f, jnp.float32(1.1754943508222875e-38))
    return -jnp.log(-jnp.log(u)) + jnp.log(w)


def _chunk_off(j):
    return jnp.where(j < FULL_STEPS, j * CHUNK, LAST_OFF)


def _body(w_hbm, wt_ref, out_ref, buf, m_acc, idx_acc, sems):
    j = pl.program_id(0)
    slot = jax.lax.rem(j, 2)

    def row_copy(off, slot_ix, r, length=BCOLS):
        src = w_hbm.at[pl.ds(off + r * BCOLS, length)]
        return pltpu.make_async_copy(src, buf.at[slot_ix, r, pl.ds(0, length)],
                                     sems.at[slot_ix])

    def start_full(off, slot_ix):
        for r in range(BROWS):
            row_copy(off, slot_ix, r).start()

    def start_last(slot_ix):
        for r in range(LAST_ROWS):
            row_copy(LAST_OFF, slot_ix, r).start()
        row_copy(LAST_OFF, slot_ix, LAST_ROWS, LAST_PART).start()

    @pl.when(j == 0)
    def _():
        start_full(0, 0)

    @pl.when(j + 1 < FULL_STEPS)
    def _():
        start_full((j + 1) * CHUNK, 1 - slot)

    @pl.when(j + 1 == FULL_STEPS)
    def _():
        start_last(1 - slot)

    @pl.when(j < FULL_STEPS)
    def _():
        for r in range(BROWS):
            row_copy(0, slot, r).wait()

    @pl.when(j == FULL_STEPS)
    def _():
        for r in range(LAST_ROWS):
            row_copy(0, slot, r).wait()
        row_copy(0, slot, LAST_ROWS, LAST_PART).wait()

    row = jax.lax.broadcasted_iota(jnp.int32, (BROWS, SCOL), 0)
    col = jax.lax.broadcasted_iota(jnp.int32, (BROWS, SCOL), 1)
    pos0 = _chunk_off(j) + row * BCOLS + col

    def step(i, carry):
        m_vec, idx_vec = carry
        pos = pos0 + i * SCOL
        w = buf[slot, :, pl.ds(i * SCOL, SCOL)]
        z = _gumbel_z(pos, w)
        z = jnp.where(pos < M_MAIN, z, -jnp.inf)
        upd = z > m_vec
        m_vec = jnp.where(upd, z, m_vec)
        idx_vec = jnp.where(upd, pos, idx_vec)
        return m_vec, idx_vec

    m0 = jnp.where(j == 0, jnp.full((BROWS, SCOL), -jnp.inf, jnp.float32),
                   m_acc[...])
    i0 = jnp.where(j == 0, jnp.full((BROWS, SCOL), 2**31 - 1, jnp.int32),
                   idx_acc[...])
    m_vec, idx_vec = jax.lax.fori_loop(
        0, BCOLS // SCOL, step, (m0, i0), unroll=8)
    m_acc[...] = m_vec
    idx_acc[...] = idx_vec

    @pl.when(j == GRID - 1)
    def _():
        m1 = jnp.max(m_vec)
        bi1 = jnp.min(jnp.where(m_vec == m1, idx_vec, jnp.int32(2**31 - 1)))
        trow = jax.lax.broadcasted_iota(jnp.int32, (8, 128), 0)
        tcol = jax.lax.broadcasted_iota(jnp.int32, (8, 128), 1)
        tpos = M_MAIN + trow * 128 + tcol
        zt = _gumbel_z(tpos, wt_ref[...])
        m2 = jnp.max(zt)
        bi2 = jnp.min(jnp.where(zt == m2, tpos, jnp.int32(2**31 - 1)))
        out_ref[0] = jnp.where(m2 > m1, bi2, bi1)


def kernel(weights):
    wt = jnp.pad(weights[M_MAIN:], (0, 8 * 128 - TAIL)).reshape(8, 128)
    idx = pl.pallas_call(
        _body,
        grid=(GRID,),
        in_specs=[pl.BlockSpec(memory_space=pltpu.HBM),
                  pl.BlockSpec((8, 128), lambda j: (0, 0))],
        out_specs=pl.BlockSpec(memory_space=pltpu.SMEM),
        out_shape=jax.ShapeDtypeStruct((1,), jnp.int32),
        scratch_shapes=[
            pltpu.VMEM((2, BROWS, BCOLS), jnp.float32),
            pltpu.VMEM((BROWS, SCOL), jnp.float32),
            pltpu.VMEM((BROWS, SCOL), jnp.int32),
            pltpu.SemaphoreType.DMA((2,)),
        ],
    )(weights, wt)
    return idx


# speedup vs baseline: 1.1382x; 1.1113x over previous
import jax
import jax.numpy as jnp
from jax.experimental import pallas as pl
from jax.experimental.pallas import tpu as pltpu

N = 1000000
ROWS, COLS = 1024, 1024
PAD = ROWS * COLS
BLOCK_ROWS = 256
GRID = ROWS // BLOCK_ROWS
STRIP = 8

_KS0 = 0
_KS1 = 42
_KS2 = _KS0 ^ _KS1 ^ 0x1BD11BDA
_ROT = ((13, 15, 26, 6), (17, 29, 16, 24))


def _rotl(x, d):
    return (x << jnp.uint32(d)) | (x >> jnp.uint32(32 - d))


def _threefry_bits(c2):
    ks = (jnp.uint32(_KS0), jnp.uint32(_KS1), jnp.uint32(_KS2))
    x0 = jnp.full(c2.shape, _KS0, jnp.uint32)
    x1 = c2 + ks[1]

    def rounds(x0, x1, rs):
        for r in rs:
            x0 = x0 + x1
            x1 = _rotl(x1, r)
            x1 = x0 ^ x1
        return x0, x1

    x0, x1 = rounds(x0, x1, _ROT[0])
    x0, x1 = x0 + ks[1], x1 + ks[2] + jnp.uint32(1)
    x0, x1 = rounds(x0, x1, _ROT[1])
    x0, x1 = x0 + ks[2], x1 + ks[0] + jnp.uint32(2)
    x0, x1 = rounds(x0, x1, _ROT[0])
    x0, x1 = x0 + ks[0], x1 + ks[1] + jnp.uint32(3)
    x0, x1 = rounds(x0, x1, _ROT[1])
    x0, x1 = x0 + ks[1], x1 + ks[2] + jnp.uint32(4)
    x0, x1 = rounds(x0, x1, _ROT[0])
    x0, x1 = x0 + ks[2], x1 + ks[0] + jnp.uint32(5)
    return x0 ^ x1


def _body(w_ref, out_ref, m_acc, idx_acc):
    j = pl.program_id(0)

    row = jax.lax.broadcasted_iota(jnp.int32, (STRIP, COLS), 0)
    col = jax.lax.broadcasted_iota(jnp.int32, (STRIP, COLS), 1)
    pos0 = j * BLOCK_ROWS * COLS + row * COLS + col

    def step(i, carry):
        m_vec, idx_vec = carry
        pos = pos0 + i * (STRIP * COLS)
        w = w_ref[pl.ds(i * STRIP, STRIP), :]
        bits = _threefry_bits(pos.astype(jnp.uint32))
        fbits = (bits >> jnp.uint32(9)) | jnp.uint32(0x3F800000)
        f = jax.lax.bitcast_convert_type(fbits, jnp.float32) - jnp.float32(1.0)
        u = jnp.maximum(f, jnp.float32(1.1754943508222875e-38))
        z = -jnp.log(-jnp.log(u)) + jnp.log(w)
        upd = z > m_vec
        m_vec = jnp.where(upd, z, m_vec)
        idx_vec = jnp.where(upd, pos, idx_vec)
        return m_vec, idx_vec

    m0 = jnp.where(j == 0, jnp.full((STRIP, COLS), -jnp.inf, jnp.float32),
                   m_acc[...])
    i0 = jnp.where(j == 0, jnp.full((STRIP, COLS), 2**31 - 1, jnp.int32),
                   idx_acc[...])
    m_vec, idx_vec = jax.lax.fori_loop(
        0, BLOCK_ROWS // STRIP, step, (m0, i0), unroll=8)
    m_acc[...] = m_vec
    idx_acc[...] = idx_vec

    @pl.when(j == GRID - 1)
    def _():
        m = jnp.max(m_vec)
        out_ref[0] = jnp.min(
            jnp.where(m_vec == m, idx_vec, jnp.int32(2**31 - 1)))


def kernel(weights):
    wp = jnp.pad(weights, (0, PAD - N)).reshape(ROWS, COLS)
    idx = pl.pallas_call(
        _body,
        grid=(GRID,),
        in_specs=[pl.BlockSpec((BLOCK_ROWS, COLS), lambda j: (j, 0))],
        out_specs=pl.BlockSpec(memory_space=pltpu.SMEM),
        out_shape=jax.ShapeDtypeStruct((1,), jnp.int32),
        scratch_shapes=[
            pltpu.VMEM((STRIP, COLS), jnp.float32),
            pltpu.VMEM((STRIP, COLS), jnp.int32),
        ],
    )(wp)
    return idx
